# baseline (speedup 1.0000x reference)
import functools

import jax
import jax.numpy as jnp
from jax import lax
from jax.experimental import pallas as pl
from jax.experimental.pallas import tpu as pltpu

OUT_F = 2048
IN_F = 2048
RANK = 2098
SLICES = 16


def _select_body(w_ref, thr_ref, xi_ref, cm_ref, st_ref):
    step = pl.program_id(0)

    @pl.when(step == 0)
    def _():
        xi = lax.bitcast_convert_type(w_ref[...], jnp.int32) & jnp.int32(
            0x7FFFFFFF
        )
        xi_ref[...] = xi
        rows = OUT_F // SLICES
        cm = xi[0:rows, :]
        for k in range(1, SLICES):
            cm = jnp.maximum(cm, xi[k * rows:(k + 1) * rows, :])
        cm_ref[...] = cm
        hi0 = jnp.max(cm) + 1
        st_ref[0] = 0
        st_ref[1] = hi0
        st_ref[2] = 0
        st_ref[3] = hi0

    @pl.when((step >= 1) & (step <= 31))
    def _():
        lo = st_ref[0]
        hi = st_ref[1]

        @pl.when(hi - lo > 1)
        def _():
            mid = lo + lax.shift_right_logical(hi - lo, 1)
            c = jnp.sum((cm_ref[...] >= mid).astype(jnp.int32))
            take = c >= RANK
            st_ref[0] = jnp.where(take, mid, lo)
            st_ref[1] = jnp.where(take, hi, mid)

    @pl.when(step == 31)
    def _():
        st_ref[2] = st_ref[0]

    @pl.when(step >= 32)
    def _():
        lo = st_ref[2]
        hi = st_ref[3]

        @pl.when(hi - lo > 1)
        def _():
            mid = lo + lax.shift_right_logical(hi - lo, 1)
            c = jnp.sum((xi_ref[...] >= mid).astype(jnp.int32))
            take = c >= RANK
            st_ref[2] = jnp.where(take, mid, lo)
            st_ref[3] = jnp.where(take, hi, mid)

    @pl.when(step == 62)
    def _():
        thr_ref[...] = jnp.full(
            (8, 128), lax.bitcast_convert_type(st_ref[2], jnp.float32)
        )


def _mask_body(w_ref, thr_ref, wm_ref):
    t = thr_ref[0, 0]
    w = w_ref[...]
    wm_ref[...] = jnp.where(jnp.abs(w) >= t, w, 0.0).astype(jnp.bfloat16)


def _matmul_body(x_ref, wm_ref, b_ref, o_ref):
    x = x_ref[...].astype(jnp.bfloat16)
    acc = lax.dot_general(
        x, wm_ref[...],
        (((1,), (1,)), ((), ())),
        preferred_element_type=jnp.float32,
    )
    o_ref[...] = acc + b_ref[...]


@jax.jit
def kernel(inputs, weight, bias):
    batch = inputs.shape[0]

    thr = pl.pallas_call(
        _select_body,
        grid=(63,),
        in_specs=[pl.BlockSpec((OUT_F, IN_F), lambda i: (0, 0))],
        out_specs=pl.BlockSpec((8, 128), lambda i: (0, 0)),
        out_shape=jax.ShapeDtypeStruct((8, 128), jnp.float32),
        scratch_shapes=[
            pltpu.VMEM((OUT_F, IN_F), jnp.int32),
            pltpu.VMEM((OUT_F // SLICES, IN_F), jnp.int32),
            pltpu.SMEM((4,), jnp.int32),
        ],
    )(weight)

    wm = pl.pallas_call(
        _mask_body,
        grid=(8,),
        in_specs=[
            pl.BlockSpec((OUT_F // 8, IN_F), lambda i: (i, 0)),
            pl.BlockSpec((8, 128), lambda i: (0, 0)),
        ],
        out_specs=pl.BlockSpec((OUT_F // 8, IN_F), lambda i: (i, 0)),
        out_shape=jax.ShapeDtypeStruct((OUT_F, IN_F), jnp.bfloat16),
    )(weight, thr)

    bm = 512
    out = pl.pallas_call(
        _matmul_body,
        grid=(batch // bm,),
        in_specs=[
            pl.BlockSpec((bm, IN_F), lambda i: (i, 0)),
            pl.BlockSpec((OUT_F, IN_F), lambda i: (0, 0)),
            pl.BlockSpec((1, OUT_F), lambda i: (0, 0)),
        ],
        out_specs=pl.BlockSpec((bm, OUT_F), lambda i: (i, 0)),
        out_shape=jax.ShapeDtypeStruct((batch, OUT_F), jnp.float32),
    )(inputs, wm, bias.reshape(1, OUT_F))
    return out


# speedup vs baseline: 1.0398x; 1.0398x over previous
import functools

import jax
import jax.numpy as jnp
from jax import lax
from jax.experimental import pallas as pl
from jax.experimental.pallas import tpu as pltpu

OUT_F = 2048
IN_F = 2048
RANK = 2098
SLICES = 64


def _select_body(w_ref, thr_ref, xi_ref, cm_ref, st_ref):
    step = pl.program_id(0)

    @pl.when(step == 0)
    def _():
        xi = lax.bitcast_convert_type(w_ref[...], jnp.int32) & jnp.int32(
            0x7FFFFFFF
        )
        xi_ref[...] = xi
        rows = OUT_F // SLICES
        cm = xi[0:rows, :]
        for k in range(1, SLICES):
            cm = jnp.maximum(cm, xi[k * rows:(k + 1) * rows, :])
        cm_ref[...] = cm
        hi0 = jnp.max(cm) + 1
        st_ref[0] = 0
        st_ref[1] = hi0
        st_ref[2] = 0
        st_ref[3] = hi0

    @pl.when((step >= 1) & (step <= 31))
    def _():
        lo = st_ref[0]
        hi = st_ref[1]

        @pl.when(hi - lo > 1)
        def _():
            mid = lo + lax.shift_right_logical(hi - lo, 1)
            c = jnp.sum((cm_ref[...] >= mid).astype(jnp.int32))
            take = c >= RANK
            st_ref[0] = jnp.where(take, mid, lo)
            st_ref[1] = jnp.where(take, hi, mid)

    @pl.when(step == 31)
    def _():
        st_ref[2] = st_ref[0]

    @pl.when(step >= 32)
    def _():
        lo = st_ref[2]
        hi = st_ref[3]

        @pl.when(hi - lo > 1)
        def _():
            mid = lo + lax.shift_right_logical(hi - lo, 1)
            c = jnp.sum((xi_ref[...] >= mid).astype(jnp.int32))
            take = c >= RANK
            st_ref[2] = jnp.where(take, mid, lo)
            st_ref[3] = jnp.where(take, hi, mid)

    @pl.when(step == 62)
    def _():
        thr_ref[...] = jnp.full(
            (8, 128), lax.bitcast_convert_type(st_ref[2], jnp.float32)
        )


def _mask_body(w_ref, thr_ref, wm_ref):
    t = thr_ref[0, 0]
    w = w_ref[...]
    wm_ref[...] = jnp.where(jnp.abs(w) >= t, w, 0.0).astype(jnp.bfloat16)


def _matmul_body(x_ref, wm_ref, b_ref, o_ref):
    x = x_ref[...].astype(jnp.bfloat16)
    acc = lax.dot_general(
        x, wm_ref[...],
        (((1,), (1,)), ((), ())),
        preferred_element_type=jnp.float32,
    )
    o_ref[...] = acc + b_ref[...]


@jax.jit
def kernel(inputs, weight, bias):
    batch = inputs.shape[0]

    thr = pl.pallas_call(
        _select_body,
        grid=(63,),
        in_specs=[pl.BlockSpec((OUT_F, IN_F), lambda i: (0, 0))],
        out_specs=pl.BlockSpec((8, 128), lambda i: (0, 0)),
        out_shape=jax.ShapeDtypeStruct((8, 128), jnp.float32),
        scratch_shapes=[
            pltpu.VMEM((OUT_F, IN_F), jnp.int32),
            pltpu.VMEM((OUT_F // SLICES, IN_F), jnp.int32),
            pltpu.SMEM((4,), jnp.int32),
        ],
    )(weight)

    wm = pl.pallas_call(
        _mask_body,
        grid=(8,),
        in_specs=[
            pl.BlockSpec((OUT_F // 8, IN_F), lambda i: (i, 0)),
            pl.BlockSpec((8, 128), lambda i: (0, 0)),
        ],
        out_specs=pl.BlockSpec((OUT_F // 8, IN_F), lambda i: (i, 0)),
        out_shape=jax.ShapeDtypeStruct((OUT_F, IN_F), jnp.bfloat16),
    )(weight, thr)

    bm = 1024
    out = pl.pallas_call(
        _matmul_body,
        grid=(batch // bm,),
        in_specs=[
            pl.BlockSpec((bm, IN_F), lambda i: (i, 0)),
            pl.BlockSpec((OUT_F, IN_F), lambda i: (0, 0)),
            pl.BlockSpec((1, OUT_F), lambda i: (0, 0)),
        ],
        out_specs=pl.BlockSpec((bm, OUT_F), lambda i: (i, 0)),
        out_shape=jax.ShapeDtypeStruct((batch, OUT_F), jnp.float32),
    )(inputs, wm, bias.reshape(1, OUT_F))
    return out


# speedup vs baseline: 1.0651x; 1.0244x over previous
import functools

import jax
import jax.numpy as jnp
from jax import lax
from jax.experimental import pallas as pl
from jax.experimental.pallas import tpu as pltpu

OUT_F = 2048
IN_F = 2048
RANK = 2098
SLICES = 64


def _select_body(w_ref, thr_ref, xi_ref, cm_ref, st_ref):
    step = pl.program_id(0)

    @pl.when(step == 0)
    def _():
        xi = lax.bitcast_convert_type(w_ref[...], jnp.int32) & jnp.int32(
            0x7FFFFFFF
        )
        xi_ref[...] = xi
        rows = OUT_F // SLICES
        cm = xi[0:rows, :]
        for k in range(1, SLICES):
            cm = jnp.maximum(cm, xi[k * rows:(k + 1) * rows, :])
        cm_ref[...] = cm
        hi0 = jnp.max(cm) + 1
        st_ref[0] = 0
        st_ref[1] = hi0
        st_ref[2] = 0
        st_ref[3] = hi0

    @pl.when((step >= 1) & (step <= 31))
    def _():
        lo = st_ref[0]
        hi = st_ref[1]

        @pl.when(hi - lo > 1)
        def _():
            mid = lo + lax.shift_right_logical(hi - lo, 1)
            c = jnp.sum((cm_ref[...] >= mid).astype(jnp.int32))
            take = c >= RANK
            st_ref[0] = jnp.where(take, mid, lo)
            st_ref[1] = jnp.where(take, hi, mid)

    @pl.when(step == 31)
    def _():
        st_ref[2] = st_ref[0]

    @pl.when(step >= 32)
    def _():
        lo = st_ref[2]
        hi = st_ref[3]

        @pl.when(hi - lo > 1)
        def _():
            mid = lo + lax.shift_right_logical(hi - lo, 1)
            c = jnp.sum((xi_ref[...] >= mid).astype(jnp.int32))
            take = c >= RANK
            st_ref[2] = jnp.where(take, mid, lo)
            st_ref[3] = jnp.where(take, hi, mid)

    @pl.when(step == 62)
    def _():
        thr_ref[...] = jnp.full(
            (8, 128), lax.bitcast_convert_type(st_ref[2], jnp.float32)
        )


def _mask_matmul_body(w_ref, thr_ref, x_ref, b_ref, o_ref, wm_ref):
    step = pl.program_id(0)

    @pl.when(step == 0)
    def _():
        t = thr_ref[0, 0]
        rows = OUT_F // 16

        def body(k, carry):
            w = w_ref[pl.ds(k * rows, rows), :]
            wm_ref[pl.ds(k * rows, rows), :] = jnp.where(
                jnp.abs(w) >= t, w, 0.0
            ).astype(jnp.bfloat16)
            return carry

        lax.fori_loop(0, 16, body, 0, unroll=False)

    @pl.when(step > 0)
    def _():
        x = x_ref[...].astype(jnp.bfloat16)
        acc = lax.dot_general(
            x, wm_ref[...],
            (((1,), (1,)), ((), ())),
            preferred_element_type=jnp.float32,
        )
        o_ref[...] = acc + b_ref[...]


@jax.jit
def kernel(inputs, weight, bias):
    batch = inputs.shape[0]

    thr = pl.pallas_call(
        _select_body,
        grid=(63,),
        in_specs=[pl.BlockSpec((OUT_F, IN_F), lambda i: (0, 0))],
        out_specs=pl.BlockSpec((8, 128), lambda i: (0, 0)),
        out_shape=jax.ShapeDtypeStruct((8, 128), jnp.float32),
        scratch_shapes=[
            pltpu.VMEM((OUT_F, IN_F), jnp.int32),
            pltpu.VMEM((OUT_F // SLICES, IN_F), jnp.int32),
            pltpu.SMEM((4,), jnp.int32),
        ],
    )(weight)

    bm = 512
    out = pl.pallas_call(
        _mask_matmul_body,
        grid=(batch // bm + 1,),
        in_specs=[
            pl.BlockSpec((OUT_F, IN_F), lambda i: (0, 0)),
            pl.BlockSpec((8, 128), lambda i: (0, 0)),
            pl.BlockSpec((bm, IN_F), lambda i: (jnp.where(i == 0, 0, i - 1), 0)),
            pl.BlockSpec((1, OUT_F), lambda i: (0, 0)),
        ],
        out_specs=pl.BlockSpec(
            (bm, OUT_F), lambda i: (jnp.where(i == 0, 0, i - 1), 0)
        ),
        out_shape=jax.ShapeDtypeStruct((batch, OUT_F), jnp.float32),
        scratch_shapes=[pltpu.VMEM((OUT_F, IN_F), jnp.bfloat16)],
    )(weight, thr, inputs, bias.reshape(1, OUT_F))
    return out
